# baseline (speedup 1.0000x reference)
import functools

import jax
import jax.numpy as jnp
from jax import lax
from jax.experimental import pallas as pl
from jax.experimental.pallas import tpu as pltpu
from jax.experimental.pallas import tpu_sc as plsc

_EMB = 64
_PAD = 128
_NC, _NS = 2, 16
_NW = _NC * _NS
_CHUNK = 256
_GRP = 1
_GROWS = _GRP * _CHUNK


@functools.partial(jax.jit, static_argnums=(2,))
def _gather(idx, table_pad, n_rows):
    nb = n_rows // _NW
    ngrp = nb // _GROWS
    mesh = plsc.VectorSubcoreMesh(core_axis_name="c", subcore_axis_name="s")

    @functools.partial(
        pl.kernel,
        out_type=jax.ShapeDtypeStruct((n_rows, _PAD), jnp.float32),
        mesh=mesh,
        scratch_types=[
            pltpu.VMEM((nb,), jnp.int32),
            pltpu.VMEM((_GROWS, _PAD), jnp.float32),
            pltpu.VMEM((_GROWS, _PAD), jnp.float32),
            pltpu.SemaphoreType.DMA,
            pltpu.SemaphoreType.DMA,
            pltpu.SemaphoreType.DMA,
            pltpu.SemaphoreType.DMA,
        ],
        compiler_params=pltpu.CompilerParams(use_tc_tiling_on_sc=True),
    )
    def gather_kernel(idx_hbm, table_hbm, out_hbm, idx_v, buf0, buf1,
                      sg0, sg1, sw0, sw1):
        wid = lax.axis_index("s") * _NC + lax.axis_index("c")
        base = wid * nb
        pltpu.sync_copy(idx_hbm.at[pl.ds(base, nb)], idx_v)

        slots = ((buf0, sg0, sw0), (buf1, sg1, sw1))

        def fire_group(g, buf, sg):
            for k in range(_GRP):
                pltpu.async_copy(
                    table_hbm.at[idx_v.at[pl.ds((g * _GRP + k) * _CHUNK, _CHUNK)]],
                    buf.at[pl.ds(k * _CHUNK, _CHUNK)],
                    sg,
                )

        def drain_group(buf, sg):
            pltpu.make_async_copy(
                table_hbm.at[pl.ds(0, _GROWS)], buf, sg
            ).wait()

        def start_write(g, buf, sw):
            pltpu.async_copy(
                buf, out_hbm.at[pl.ds(base + g * _GROWS, _GROWS)], sw
            )

        def wait_write(buf, sw):
            pltpu.make_async_copy(
                buf, out_hbm.at[pl.ds(base, _GROWS)], sw
            ).wait()

        fire_group(0, buf0, sg0)

        @pl.loop(0, ngrp // 2)
        def _(g2):
            for p in range(2):
                g = g2 * 2 + p
                buf, sg, sw = slots[p]
                obuf, osg, osw = slots[1 - p]
                drain_group(buf, sg)
                @pl.when(g >= 1)
                def _():
                    wait_write(obuf, osw)

                @pl.when(g + 1 < ngrp)
                def _():
                    fire_group(g + 1, obuf, osg)

                start_write(g, buf, sw)

        lbuf, _lsg, lsw = slots[(ngrp - 1) % 2]
        wait_write(lbuf, lsw)

    return gather_kernel(idx, table_pad)


def kernel(indices, table):
    b, l = indices.shape
    n = b * l
    v, d = table.shape
    table_pad = jnp.pad(table, ((0, 0), (0, _PAD - d)))
    idx = indices.reshape(n)
    out = _gather(idx, table_pad, n)
    return out[:, :d].reshape(b, l, d)


# speedup vs baseline: 1.0042x; 1.0042x over previous
import functools

import jax
import jax.numpy as jnp
from jax import lax
from jax.experimental import pallas as pl
from jax.experimental.pallas import tpu as pltpu
from jax.experimental.pallas import tpu_sc as plsc

_EMB = 64
_PAD = 128
_NC, _NS = 2, 16
_NW = _NC * _NS
_CHUNK = 128
_GRP = 2
_GROWS = _GRP * _CHUNK


@functools.partial(jax.jit, static_argnums=(2,))
def _gather(idx, table_pad, n_rows):
    nb = n_rows // _NW
    ngrp = nb // _GROWS
    mesh = plsc.VectorSubcoreMesh(core_axis_name="c", subcore_axis_name="s")

    @functools.partial(
        pl.kernel,
        out_type=jax.ShapeDtypeStruct((n_rows, _PAD), jnp.float32),
        mesh=mesh,
        scratch_types=[
            pltpu.VMEM((nb,), jnp.int32),
            pltpu.VMEM((_GROWS, _PAD), jnp.float32),
            pltpu.VMEM((_GROWS, _PAD), jnp.float32),
            pltpu.SemaphoreType.DMA,
            pltpu.SemaphoreType.DMA,
            pltpu.SemaphoreType.DMA,
            pltpu.SemaphoreType.DMA,
        ],
        compiler_params=pltpu.CompilerParams(use_tc_tiling_on_sc=True),
    )
    def gather_kernel(idx_hbm, table_hbm, out_hbm, idx_v, buf0, buf1,
                      sg0, sg1, sw0, sw1):
        wid = lax.axis_index("s") * _NC + lax.axis_index("c")
        base = wid * nb
        pltpu.sync_copy(idx_hbm.at[pl.ds(base, nb)], idx_v)

        slots = ((buf0, sg0, sw0), (buf1, sg1, sw1))

        def fire_group(g, buf, sg):
            for k in range(_GRP):
                pltpu.async_copy(
                    table_hbm.at[idx_v.at[pl.ds((g * _GRP + k) * _CHUNK, _CHUNK)]],
                    buf.at[pl.ds(k * _CHUNK, _CHUNK)],
                    sg,
                )

        def drain_group(buf, sg):
            pltpu.make_async_copy(
                table_hbm.at[pl.ds(0, _GROWS)], buf, sg
            ).wait()

        def start_write(g, buf, sw):
            pltpu.async_copy(
                buf, out_hbm.at[pl.ds(base + g * _GROWS, _GROWS)], sw
            )

        def wait_write(buf, sw):
            pltpu.make_async_copy(
                buf, out_hbm.at[pl.ds(base, _GROWS)], sw
            ).wait()

        fire_group(0, buf0, sg0)

        @pl.loop(0, ngrp // 2)
        def _(g2):
            for p in range(2):
                g = g2 * 2 + p
                buf, sg, sw = slots[p]
                obuf, osg, osw = slots[1 - p]
                drain_group(buf, sg)
                @pl.when(g >= 1)
                def _():
                    wait_write(obuf, osw)

                @pl.when(g + 1 < ngrp)
                def _():
                    fire_group(g + 1, obuf, osg)

                start_write(g, buf, sw)

        lbuf, _lsg, lsw = slots[(ngrp - 1) % 2]
        wait_write(lbuf, lsw)

    return gather_kernel(idx, table_pad)


def kernel(indices, table):
    b, l = indices.shape
    n = b * l
    v, d = table.shape
    table_pad = jnp.pad(table, ((0, 0), (0, _PAD - d)))
    idx = indices.reshape(n)
    out = _gather(idx, table_pad, n)
    return out[:, :d].reshape(b, l, d)
